# baseline (speedup 1.0000x reference)
import functools

import jax
import jax.numpy as jnp
from jax import lax
from jax.experimental import pallas as pl
from jax.experimental.pallas import tpu as pltpu
from jax.experimental.pallas import tpu_sc as plsc

E = 128
D = 128
NC = 2
NS = 16
L = 16
NW = NC * NS
CH = 64


def _project_tables_body(t_ref, w_ref, b_ref, out_ref):
    tp = lax.dot_general(
        t_ref[...], w_ref[...], (((1,), (1,)), ((), ())),
        preferred_element_type=jnp.float32)
    rows = lax.broadcasted_iota(jnp.int32, (4 * E, 1), 0)
    in_w_seg = (rows >= 2 * E) & (rows < 3 * E)
    out_ref[...] = tp + jnp.where(in_w_seg, b_ref[...], jnp.float32(0.0))


def _project_tables(tables, w, b):
    return pl.pallas_call(
        _project_tables_body,
        out_shape=jax.ShapeDtypeStruct((4 * E, D), jnp.float32),
    )(tables, w, b.reshape(1, D))


def _sc_lookup_body(n_rows, n_per_batch, n_scales, coords_hbm, tp_hbm,
                    scales_hbm, out_hbm,
                    box0, box1, idx0, idx1, tp_v, o0, o1, sc_v,
                    isem0, isem1, osem0, osem1):
    rpw = n_rows // NW
    n_full = rpw // CH
    tail = rpw - n_full * CH
    wid = lax.axis_index("s") * NC + lax.axis_index("c")
    base = wid * rpw
    batch = base // n_per_batch

    box = (box0, box1)
    idx = (idx0, idx1)
    out = (o0, o1)
    isem = (isem0, isem1)
    osem = (osem0, osem1)

    pltpu.sync_copy(tp_hbm, tp_v)
    pltpu.sync_copy(scales_hbm, sc_v.at[pl.ds(0, n_scales)])
    iota = lax.broadcasted_iota(jnp.int32, (L,), 0)
    h_img = plsc.load_gather(sc_v, [jnp.full((L,), 2 * batch, jnp.int32)])
    w_img = plsc.load_gather(sc_v, [jnp.full((L,), 2 * batch + 1, jnp.int32)])
    ef = jnp.float32(E)
    emax = jnp.float32(E - 1)

    def in_slice(row0, rows):
        return coords_hbm.at[pl.ds(row0 * 8, rows * 8)]

    def in_start(row0, b, rows=CH):
        pltpu.async_copy(in_slice(row0, rows), box[b].at[pl.ds(0, rows * 8)],
                         isem[b])

    def in_wait(row0, b, rows=CH):
        pltpu.make_async_copy(in_slice(row0, rows),
                              box[b].at[pl.ds(0, rows * 8)], isem[b]).wait()

    def idx_compute(b, rows=CH):
        for j in range(rows // L):
            rows_k = (j * L + iota) * 8

            def coord(k):
                return plsc.load_gather(box[b], [rows_k + k])

            x0, x1, x2, x3 = coord(0), coord(2), coord(4), coord(6)
            y0, y1, y2, y3 = coord(1), coord(3), coord(5), coord(7)
            xminf = jnp.minimum(jnp.minimum(x0, x1), jnp.minimum(x2, x3))
            xmaxf = jnp.maximum(jnp.maximum(x0, x1), jnp.maximum(x2, x3))
            yminf = jnp.minimum(jnp.minimum(y0, y1), jnp.minimum(y2, y3))
            ymaxf = jnp.maximum(jnp.maximum(y0, y1), jnp.maximum(y2, y3))

            def to_idx(v, denom):
                scaled = (v / denom) * ef
                return jnp.clip(scaled, jnp.float32(0.0), emax).astype(jnp.int32)

            ixmin = to_idx(xminf, w_img)
            ixmax = to_idx(xmaxf, w_img)
            iymin = to_idx(yminf, h_img)
            iymax = to_idx(ymaxf, h_img)
            sl = pl.ds(j * L, L)
            idx[b][0, sl] = ixmin
            idx[b][1, sl] = iymin + E
            idx[b][2, sl] = ixmax
            idx[b][3, sl] = iymax + E
            idx[b][4, sl] = (ixmax - ixmin) + 2 * E
            idx[b][5, sl] = (iymax - iymin) + 3 * E

    offs = [w * L + iota for w in range(D // L)]
    _dnums = lax.GatherDimensionNumbers(
        offset_dims=(), collapsed_slice_dims=(0,), start_index_map=(0,))

    def vtake(v, lane):
        return lax.gather(v, lane[:, None], _dnums, (1,),
                          mode=lax.GatherScatterMode.PROMISE_IN_BOUNDS)

    def accumulate(b, rows=CH):
        iv = idx[b]
        ov = out[b]

        @pl.loop(0, rows // L)
        def _acc(j):
            r0 = j * L
            ivec = [iv[t, pl.ds(r0, L)] * D for t in range(6)]
            for l in range(L):
                lane = jnp.full((L,), l, jnp.int32)
                bases = [vtake(ivec[t], lane) for t in range(6)]
                r = r0 + l
                for w in range(D // L):
                    s = pl.ds(w * L, L)

                    def gl(t):
                        return plsc.load_gather(tp_v, [bases[t] + offs[w]])

                    acc = ((gl(0) + gl(1)) + (gl(2) + gl(3))) + (gl(4) + gl(5))
                    ov[r, s] = jnp.maximum(acc, jnp.float32(0.0))

    def out_slice(row0, rows):
        return out_hbm.at[pl.ds(row0, rows)]

    def out_start(row0, b, rows=CH):
        pltpu.async_copy(out[b].at[pl.ds(0, rows)], out_slice(row0, rows),
                         osem[b])

    def out_wait(row0, b, rows=CH):
        pltpu.make_async_copy(out[b].at[pl.ds(0, rows)],
                              out_slice(row0, rows), osem[b]).wait()

    pltpu.sync_copy(in_slice(base, CH), box[0].at[pl.ds(0, CH * 8)])
    idx_compute(0)
    in_start(base + CH, 1)

    def step(g, p):
        @pl.when(g + 1 < n_full)
        def _():
            in_wait(base + (g + 1) * CH, 1 - p)
            idx_compute(1 - p)

            @pl.when(g + 2 < n_full)
            def _():
                in_start(base + (g + 2) * CH, p)

        @pl.when(g >= 2)
        def _():
            out_wait(base, p)

        accumulate(p)
        out_start(base + g * CH, p)

    @pl.loop(0, n_full // 2)
    def _pair(q):
        step(2 * q, 0)
        step(2 * q + 1, 1)

    if n_full % 2:
        step(n_full - 1, (n_full - 1) % 2)

    out_wait(base, 0)
    out_wait(base, 1)

    if tail:
        trow0 = base + n_full * CH
        pltpu.sync_copy(in_slice(trow0, tail), box[0].at[pl.ds(0, tail * 8)])
        idx_compute(0, tail)
        accumulate(0, tail)
        out_start(trow0, 0, tail)
        out_wait(trow0, 0, tail)


def kernel(boxes, img_shapes, x_emb, y_emb, w_emb, h_emb, W, b):
    B, N, K = boxes.shape
    n_rows = B * N
    tables = jnp.concatenate([x_emb, y_emb, w_emb, h_emb], axis=0)
    tp = _project_tables(tables, W, b)

    boxes2 = boxes.reshape(n_rows * K)

    mesh = plsc.VectorSubcoreMesh(core_axis_name="c", subcore_axis_name="s")
    body = functools.partial(_sc_lookup_body, n_rows, N, B * 2)
    out = pl.kernel(
        body,
        out_type=jax.ShapeDtypeStruct((n_rows, D), jnp.float32),
        mesh=mesh,
        compiler_params=pltpu.CompilerParams(needs_layout_passes=False),
        scratch_types=[
            pltpu.VMEM((CH * K,), jnp.float32),
            pltpu.VMEM((CH * K,), jnp.float32),
            pltpu.VMEM((6, CH), jnp.int32),
            pltpu.VMEM((6, CH), jnp.int32),
            pltpu.VMEM((4 * E * D,), jnp.float32),
            pltpu.VMEM((CH, D), jnp.float32),
            pltpu.VMEM((CH, D), jnp.float32),
            pltpu.VMEM((max(B * 2, 128),), jnp.float32),
            pltpu.SemaphoreType.DMA,
            pltpu.SemaphoreType.DMA,
            pltpu.SemaphoreType.DMA,
            pltpu.SemaphoreType.DMA,
        ],
    )(boxes2, tp.reshape(4 * E * D), img_shapes.reshape(B * 2))
    return out.reshape(B, N, D)


# speedup vs baseline: 2.1533x; 2.1533x over previous
import functools

import jax
import jax.numpy as jnp
from jax import lax
from jax.experimental import pallas as pl
from jax.experimental.pallas import tpu as pltpu
from jax.experimental.pallas import tpu_sc as plsc

E = 128
D = 128
NC = 2
NS = 16
L = 16
NW = NC * NS
CH = 64
BS = 8
RT = 1000


def _project_tables_body(t_ref, w_ref, b_ref, out_ref):
    tp = lax.dot_general(
        t_ref[...], w_ref[...], (((1,), (1,)), ((), ())),
        preferred_element_type=jnp.float32)
    rows = lax.broadcasted_iota(jnp.int32, (4 * E, 1), 0)
    in_w_seg = (rows >= 2 * E) & (rows < 3 * E)
    out_ref[...] = tp + jnp.where(in_w_seg, b_ref[...], jnp.float32(0.0))


def _project_tables(tables, w, b):
    return pl.pallas_call(
        _project_tables_body,
        out_shape=jax.ShapeDtypeStruct((4 * E, D), jnp.float32),
    )(tables, w, b.reshape(1, D))


def _tc_half_body(xs_ref, ys_ref, scales_ref, tp_ref, o_ref):
    b = pl.program_id(0)
    w_img = scales_ref[b, 1]
    h_img = scales_ref[b, 0]
    xs = xs_ref[0]
    ys = ys_ref[0]
    xminf = jnp.min(xs, axis=1, keepdims=True)
    xmaxf = jnp.max(xs, axis=1, keepdims=True)
    yminf = jnp.min(ys, axis=1, keepdims=True)
    ymaxf = jnp.max(ys, axis=1, keepdims=True)

    def to_idx(v, denom):
        scaled = (v / denom) * jnp.float32(E)
        return jnp.clip(scaled, jnp.float32(0.0),
                        jnp.float32(E - 1)).astype(jnp.int32)

    ixmin = to_idx(xminf, w_img)
    ixmax = to_idx(xmaxf, w_img)
    iymin = to_idx(yminf, h_img)
    iymax = to_idx(ymaxf, h_img)
    iota = lax.broadcasted_iota(jnp.int32, (RT, E), 1)

    def oh(i):
        return (iota == i).astype(jnp.bfloat16)

    c = jnp.concatenate(
        [oh(ixmin) + oh(ixmax), oh(iymin) + oh(iymax),
         oh(ixmax - ixmin), oh(iymax - iymin)], axis=1)
    acc = jnp.dot(c, tp_ref[...], preferred_element_type=jnp.float32)
    o_ref[0] = jnp.maximum(acc, jnp.float32(0.0))


def _tc_half(bx, by, img_shapes, tp_bf):
    nb = bx.shape[0]
    nblk = bx.shape[1] // RT
    return pl.pallas_call(
        _tc_half_body,
        grid=(nb, nblk),
        in_specs=[
            pl.BlockSpec((1, RT, 4), lambda b, i: (b, i, 0)),
            pl.BlockSpec((1, RT, 4), lambda b, i: (b, i, 0)),
            pl.BlockSpec(memory_space=pltpu.SMEM),
            pl.BlockSpec((4 * E, D), lambda b, i: (0, 0)),
        ],
        out_specs=pl.BlockSpec((1, RT, D), lambda b, i: (b, i, 0)),
        out_shape=jax.ShapeDtypeStruct((nb, bx.shape[1], D), jnp.float32),
    )(bx, by, img_shapes, tp_bf)


def _sc_lookup_body(s0, n_sc_rows, n_per_batch, n_scales, coords_hbm, tp_hbm,
                    scales_hbm, out_hbm,
                    box0, box1, idx0, idx1, g0, g1, o0, o1, sc_v,
                    isem0, isem1, gsem0, gsem1, osem0, osem1):
    n_chunks = n_sc_rows // CH
    nb_floor = n_chunks // NW
    extra = n_chunks - nb_floor * NW
    wid = lax.axis_index("s") * NC + lax.axis_index("c")
    n_w = nb_floor + jnp.where(wid < extra, 1, 0)

    box = (box0, box1)
    idx = (idx0, idx1)
    gat = (g0, g1)
    out = (o0, o1)
    isem = (isem0, isem1)
    gsem = (gsem0, gsem1)
    osem = (osem0, osem1)

    pltpu.sync_copy(scales_hbm, sc_v.at[pl.ds(0, n_scales)])
    iota = lax.broadcasted_iota(jnp.int32, (L,), 0)
    ef = jnp.float32(E)
    emax = jnp.float32(E - 1)

    def row0(q):
        return (q * NW + wid) * CH

    def in_slice(r0):
        return coords_hbm.at[pl.ds(r0 * 8, CH * 8)]

    def in_start(r0, b):
        pltpu.async_copy(in_slice(r0), box[b], isem[b])

    def in_wait(r0, b):
        pltpu.make_async_copy(in_slice(r0), box[b], isem[b]).wait()

    def idx_compute(b, r0):
        for j in range(CH // L):
            batchg = (s0 + r0 + j * L) // n_per_batch
            h_img = plsc.load_gather(
                sc_v, [jnp.full((L,), 2 * batchg, jnp.int32)])
            w_img = plsc.load_gather(
                sc_v, [jnp.full((L,), 2 * batchg + 1, jnp.int32)])
            rows_k = (j * L + iota) * 8

            def coord(k):
                return plsc.load_gather(box[b], [rows_k + k])

            x0, x1, x2, x3 = coord(0), coord(2), coord(4), coord(6)
            y0, y1, y2, y3 = coord(1), coord(3), coord(5), coord(7)
            xminf = jnp.minimum(jnp.minimum(x0, x1), jnp.minimum(x2, x3))
            xmaxf = jnp.maximum(jnp.maximum(x0, x1), jnp.maximum(x2, x3))
            yminf = jnp.minimum(jnp.minimum(y0, y1), jnp.minimum(y2, y3))
            ymaxf = jnp.maximum(jnp.maximum(y0, y1), jnp.maximum(y2, y3))

            def to_idx(v, denom):
                scaled = (v / denom) * ef
                return jnp.clip(scaled, jnp.float32(0.0), emax).astype(jnp.int32)

            ixmin = to_idx(xminf, w_img)
            ixmax = to_idx(xmaxf, w_img)
            iymin = to_idx(yminf, h_img)
            iymax = to_idx(ymaxf, h_img)
            sl = pl.ds(j * L, L)
            idx[b][0, sl] = ixmin
            idx[b][1, sl] = iymin + E
            idx[b][2, sl] = ixmax
            idx[b][3, sl] = iymax + E
            idx[b][4, sl] = (ixmax - ixmin) + 2 * E
            idx[b][5, sl] = (iymax - iymin) + 3 * E

    def gather_start(b):
        for t in range(6):
            pltpu.async_copy(tp_hbm.at[idx[b].at[t]], gat[b].at[t], gsem[b])

    def gather_wait(b):
        for t in range(6):
            pltpu.make_async_copy(tp_hbm.at[idx[b].at[t]], gat[b].at[t],
                                  gsem[b]).wait()

    def accumulate(b):
        gv = gat[b]
        ov = out[b]

        @pl.loop(0, CH, unroll=2)
        def _acc(r):
            for w in range(D // L):
                s = pl.ds(w * L, L)
                acc = ((gv[0, r, s] + gv[1, r, s])
                       + (gv[2, r, s] + gv[3, r, s])
                       + (gv[4, r, s] + gv[5, r, s]))
                ov[r, s] = jnp.maximum(acc, jnp.float32(0.0))

    def out_start(r0, b):
        pltpu.async_copy(out[b], out_hbm.at[pl.ds(r0, CH)], osem[b])

    def out_wait(b):
        pltpu.make_async_copy(out[b], out_hbm.at[pl.ds(0, CH)],
                              osem[b]).wait()

    pltpu.sync_copy(in_slice(row0(0)), box[0])
    idx_compute(0, row0(0))
    gather_start(0)
    in_start(row0(1), 1)

    def step(q, p):
        @pl.when(q + 1 < n_w)
        def _():
            in_wait(row0(q + 1), 1 - p)
            idx_compute(1 - p, row0(q + 1))
            gather_start(1 - p)

            @pl.when(q + 2 < n_w)
            def _():
                in_start(row0(q + 2), p)

        gather_wait(p)

        @pl.when(q >= 2)
        def _():
            out_wait(p)

        accumulate(p)
        out_start(row0(q), p)

    @pl.loop(0, nb_floor // 2)
    def _pair(qq):
        step(2 * qq, 0)
        step(2 * qq + 1, 1)

    @pl.when(wid < extra)
    def _():
        step(nb_floor, 0)

    out_wait(0)
    out_wait(1)


def kernel(boxes, img_shapes, x_emb, y_emb, w_emb, h_emb, W, b):
    B, N, K = boxes.shape
    n_rows = B * N
    tables = jnp.concatenate([x_emb, y_emb, w_emb, h_emb], axis=0)
    tp = _project_tables(tables, W, b)

    s0 = BS * N
    n_sc = n_rows - s0
    boxes_sc = boxes.reshape(n_rows, K)[s0:].reshape(n_sc * K)

    mesh = plsc.VectorSubcoreMesh(core_axis_name="c", subcore_axis_name="s")
    body = functools.partial(_sc_lookup_body, s0, n_sc, N, B * 2)
    out_sc = pl.kernel(
        body,
        out_type=jax.ShapeDtypeStruct((n_sc, D), jnp.float32),
        mesh=mesh,
        compiler_params=pltpu.CompilerParams(needs_layout_passes=False),
        scratch_types=[
            pltpu.VMEM((CH * K,), jnp.float32),
            pltpu.VMEM((CH * K,), jnp.float32),
            pltpu.VMEM((6, CH), jnp.int32),
            pltpu.VMEM((6, CH), jnp.int32),
            pltpu.VMEM((6, CH, D), jnp.float32),
            pltpu.VMEM((6, CH, D), jnp.float32),
            pltpu.VMEM((CH, D), jnp.float32),
            pltpu.VMEM((CH, D), jnp.float32),
            pltpu.VMEM((max(B * 2, 128),), jnp.float32),
            pltpu.SemaphoreType.DMA,
            pltpu.SemaphoreType.DMA,
            pltpu.SemaphoreType.DMA,
            pltpu.SemaphoreType.DMA,
            pltpu.SemaphoreType.DMA,
            pltpu.SemaphoreType.DMA,
        ],
    )(boxes_sc, tp, img_shapes.reshape(B * 2))

    bx = boxes[:BS, :, 0::2]
    by = boxes[:BS, :, 1::2]
    out_tc = _tc_half(bx, by, img_shapes, tp.astype(jnp.bfloat16))

    return jnp.concatenate(
        [out_tc, out_sc.reshape(B - BS, N, D)], axis=0)


# speedup vs baseline: 2.7148x; 1.2607x over previous
import functools

import jax
import jax.numpy as jnp
from jax import lax
from jax.experimental import pallas as pl
from jax.experimental.pallas import tpu as pltpu
from jax.experimental.pallas import tpu_sc as plsc

E = 128
D = 128
NC = 2
NS = 16
L = 16
NW = NC * NS
CH = 64
BS = 12
RT = 1000


def _project_tables_body(t_ref, w_ref, b_ref, out_ref):
    tp = lax.dot_general(
        t_ref[...], w_ref[...], (((1,), (1,)), ((), ())),
        preferred_element_type=jnp.float32)
    rows = lax.broadcasted_iota(jnp.int32, (4 * E, 1), 0)
    in_w_seg = (rows >= 2 * E) & (rows < 3 * E)
    out_ref[...] = tp + jnp.where(in_w_seg, b_ref[...], jnp.float32(0.0))


def _project_tables(tables, w, b):
    return pl.pallas_call(
        _project_tables_body,
        out_shape=jax.ShapeDtypeStruct((4 * E, D), jnp.float32),
    )(tables, w, b.reshape(1, D))


def _tc_half_body(xs_ref, ys_ref, scales_ref, tp_ref, o_ref):
    b = pl.program_id(0)
    w_img = scales_ref[b, 1]
    h_img = scales_ref[b, 0]
    xs = xs_ref[0]
    ys = ys_ref[0]
    xminf = jnp.min(xs, axis=1, keepdims=True)
    xmaxf = jnp.max(xs, axis=1, keepdims=True)
    yminf = jnp.min(ys, axis=1, keepdims=True)
    ymaxf = jnp.max(ys, axis=1, keepdims=True)

    def to_idx(v, denom):
        scaled = (v / denom) * jnp.float32(E)
        return jnp.clip(scaled, jnp.float32(0.0),
                        jnp.float32(E - 1)).astype(jnp.int32)

    ixmin = to_idx(xminf, w_img)
    ixmax = to_idx(xmaxf, w_img)
    iymin = to_idx(yminf, h_img)
    iymax = to_idx(ymaxf, h_img)
    iota = lax.broadcasted_iota(jnp.int32, (RT, E), 1)

    def oh(i):
        return (iota == i).astype(jnp.bfloat16)

    c = jnp.concatenate(
        [oh(ixmin) + oh(ixmax), oh(iymin) + oh(iymax),
         oh(ixmax - ixmin), oh(iymax - iymin)], axis=1)
    acc = jnp.dot(c, tp_ref[...], preferred_element_type=jnp.float32)
    o_ref[0] = jnp.maximum(acc, jnp.float32(0.0))


def _tc_half(bx, by, img_shapes, tp_bf):
    nb = bx.shape[0]
    nblk = bx.shape[1] // RT
    return pl.pallas_call(
        _tc_half_body,
        grid=(nb, nblk),
        in_specs=[
            pl.BlockSpec((1, RT, 4), lambda b, i: (b, i, 0)),
            pl.BlockSpec((1, RT, 4), lambda b, i: (b, i, 0)),
            pl.BlockSpec(memory_space=pltpu.SMEM),
            pl.BlockSpec((4 * E, D), lambda b, i: (0, 0)),
        ],
        out_specs=pl.BlockSpec((1, RT, D), lambda b, i: (b, i, 0)),
        out_shape=jax.ShapeDtypeStruct((nb, bx.shape[1], D), jnp.float32),
    )(bx, by, img_shapes, tp_bf)


def _sc_lookup_body(s0, n_sc_rows, n_per_batch, n_scales, coords_hbm, tp_hbm,
                    scales_hbm, out_hbm,
                    box0, box1, idx0, idx1, g0, g1, o0, o1, sc_v,
                    isem0, isem1, gsem0, gsem1, osem0, osem1):
    n_chunks = n_sc_rows // CH
    nb_floor = n_chunks // NW
    extra = n_chunks - nb_floor * NW
    wid = lax.axis_index("s") * NC + lax.axis_index("c")
    n_w = nb_floor + jnp.where(wid < extra, 1, 0)

    box = (box0, box1)
    idx = (idx0, idx1)
    gat = (g0, g1)
    out = (o0, o1)
    isem = (isem0, isem1)
    gsem = (gsem0, gsem1)
    osem = (osem0, osem1)

    pltpu.sync_copy(scales_hbm, sc_v.at[pl.ds(0, n_scales)])
    iota = lax.broadcasted_iota(jnp.int32, (L,), 0)
    ef = jnp.float32(E)
    emax = jnp.float32(E - 1)

    def row0(q):
        return (q * NW + wid) * CH

    def in_slice(r0):
        return coords_hbm.at[pl.ds(r0 * 8, CH * 8)]

    def in_start(r0, b):
        pltpu.async_copy(in_slice(r0), box[b], isem[b])

    def in_wait(r0, b):
        pltpu.make_async_copy(in_slice(r0), box[b], isem[b]).wait()

    def idx_compute(b, r0):
        for j in range(CH // L):
            batchg = (s0 + r0 + j * L) // n_per_batch
            h_img = plsc.load_gather(
                sc_v, [jnp.full((L,), 2 * batchg, jnp.int32)])
            w_img = plsc.load_gather(
                sc_v, [jnp.full((L,), 2 * batchg + 1, jnp.int32)])
            rows_k = (j * L + iota) * 8

            def coord(k):
                return plsc.load_gather(box[b], [rows_k + k])

            x0, x1, x2, x3 = coord(0), coord(2), coord(4), coord(6)
            y0, y1, y2, y3 = coord(1), coord(3), coord(5), coord(7)
            xminf = jnp.minimum(jnp.minimum(x0, x1), jnp.minimum(x2, x3))
            xmaxf = jnp.maximum(jnp.maximum(x0, x1), jnp.maximum(x2, x3))
            yminf = jnp.minimum(jnp.minimum(y0, y1), jnp.minimum(y2, y3))
            ymaxf = jnp.maximum(jnp.maximum(y0, y1), jnp.maximum(y2, y3))

            def to_idx(v, denom):
                scaled = (v / denom) * ef
                return jnp.clip(scaled, jnp.float32(0.0), emax).astype(jnp.int32)

            ixmin = to_idx(xminf, w_img)
            ixmax = to_idx(xmaxf, w_img)
            iymin = to_idx(yminf, h_img)
            iymax = to_idx(ymaxf, h_img)
            sl = pl.ds(j * L, L)
            idx[b][0, sl] = ixmin
            idx[b][1, sl] = iymin + E
            idx[b][2, sl] = ixmax
            idx[b][3, sl] = iymax + E
            idx[b][4, sl] = (ixmax - ixmin) + 2 * E
            idx[b][5, sl] = (iymax - iymin) + 3 * E

    def gather_start(b):
        for t in range(6):
            pltpu.async_copy(tp_hbm.at[idx[b].at[t]], gat[b].at[t], gsem[b])

    def gather_wait(b):
        for t in range(6):
            pltpu.make_async_copy(tp_hbm.at[idx[b].at[t]], gat[b].at[t],
                                  gsem[b]).wait()

    def accumulate(b):
        gv = gat[b]
        ov = out[b]

        @pl.loop(0, CH, unroll=2)
        def _acc(r):
            for w in range(D // L):
                s = pl.ds(w * L, L)
                acc = ((gv[0, r, s] + gv[1, r, s])
                       + (gv[2, r, s] + gv[3, r, s])
                       + (gv[4, r, s] + gv[5, r, s]))
                ov[r, s] = jnp.maximum(acc, jnp.float32(0.0))

    def out_start(r0, b):
        pltpu.async_copy(out[b], out_hbm.at[pl.ds(r0, CH)], osem[b])

    def out_wait(b):
        pltpu.make_async_copy(out[b], out_hbm.at[pl.ds(0, CH)],
                              osem[b]).wait()

    pltpu.sync_copy(in_slice(row0(0)), box[0])
    idx_compute(0, row0(0))
    gather_start(0)
    in_start(row0(1), 1)

    def step(q, p):
        @pl.when(q + 1 < n_w)
        def _():
            in_wait(row0(q + 1), 1 - p)
            idx_compute(1 - p, row0(q + 1))
            gather_start(1 - p)

            @pl.when(q + 2 < n_w)
            def _():
                in_start(row0(q + 2), p)

        gather_wait(p)

        @pl.when(q >= 2)
        def _():
            out_wait(p)

        accumulate(p)
        out_start(row0(q), p)

    @pl.loop(0, nb_floor // 2)
    def _pair(qq):
        step(2 * qq, 0)
        step(2 * qq + 1, 1)

    if nb_floor % 2:
        step(nb_floor - 1, (nb_floor - 1) % 2)

    @pl.when(wid < extra)
    def _():
        step(nb_floor, nb_floor % 2)

    out_wait(0)
    out_wait(1)


def kernel(boxes, img_shapes, x_emb, y_emb, w_emb, h_emb, W, b):
    B, N, K = boxes.shape
    n_rows = B * N
    tables = jnp.concatenate([x_emb, y_emb, w_emb, h_emb], axis=0)
    tp = _project_tables(tables, W, b)

    s0 = BS * N
    n_sc = n_rows - s0
    boxes_sc = boxes.reshape(n_rows, K)[s0:].reshape(n_sc * K)

    bx = boxes[:BS, :, 0::2]
    by = boxes[:BS, :, 1::2]
    out_tc = _tc_half(bx, by, img_shapes, tp.astype(jnp.bfloat16))

    mesh = plsc.VectorSubcoreMesh(core_axis_name="c", subcore_axis_name="s")
    body = functools.partial(_sc_lookup_body, s0, n_sc, N, B * 2)
    out_sc = pl.kernel(
        body,
        out_type=jax.ShapeDtypeStruct((n_sc, D), jnp.float32),
        mesh=mesh,
        compiler_params=pltpu.CompilerParams(needs_layout_passes=False),
        scratch_types=[
            pltpu.VMEM((CH * K,), jnp.float32),
            pltpu.VMEM((CH * K,), jnp.float32),
            pltpu.VMEM((6, CH), jnp.int32),
            pltpu.VMEM((6, CH), jnp.int32),
            pltpu.VMEM((6, CH, D), jnp.float32),
            pltpu.VMEM((6, CH, D), jnp.float32),
            pltpu.VMEM((CH, D), jnp.float32),
            pltpu.VMEM((CH, D), jnp.float32),
            pltpu.VMEM((max(B * 2, 128),), jnp.float32),
            pltpu.SemaphoreType.DMA,
            pltpu.SemaphoreType.DMA,
            pltpu.SemaphoreType.DMA,
            pltpu.SemaphoreType.DMA,
            pltpu.SemaphoreType.DMA,
            pltpu.SemaphoreType.DMA,
        ],
    )(boxes_sc, tp, img_shapes.reshape(B * 2))

    return jnp.concatenate(
        [out_tc, out_sc.reshape(B - BS, N, D)], axis=0)


# speedup vs baseline: 3.1808x; 1.1717x over previous
import functools

import jax
import jax.numpy as jnp
from jax import lax
from jax.experimental import pallas as pl
from jax.experimental.pallas import tpu as pltpu
from jax.experimental.pallas import tpu_sc as plsc

E = 128
D = 128
NC = 2
NS = 16
L = 16
NW = NC * NS
CH = 64
BS = 14
RT = 2000


def _project_tables_body(t_ref, w_ref, b_ref, out_ref):
    tp = lax.dot_general(
        t_ref[...], w_ref[...], (((1,), (1,)), ((), ())),
        preferred_element_type=jnp.float32)
    rows = lax.broadcasted_iota(jnp.int32, (4 * E, 1), 0)
    in_w_seg = (rows >= 2 * E) & (rows < 3 * E)
    out_ref[...] = tp + jnp.where(in_w_seg, b_ref[...], jnp.float32(0.0))


def _project_tables(tables, w, b):
    return pl.pallas_call(
        _project_tables_body,
        out_shape=jax.ShapeDtypeStruct((4 * E, D), jnp.float32),
    )(tables, w, b.reshape(1, D))


def _tc_half_body(xs_ref, ys_ref, scales_ref, tp_ref, o_ref):
    b = pl.program_id(0)
    w_img = scales_ref[b, 1]
    h_img = scales_ref[b, 0]
    xs = xs_ref[0]
    ys = ys_ref[0]
    xminf = jnp.min(xs, axis=1, keepdims=True)
    xmaxf = jnp.max(xs, axis=1, keepdims=True)
    yminf = jnp.min(ys, axis=1, keepdims=True)
    ymaxf = jnp.max(ys, axis=1, keepdims=True)

    def to_idx(v, denom):
        scaled = (v / denom) * jnp.float32(E)
        return jnp.clip(scaled, jnp.float32(0.0),
                        jnp.float32(E - 1)).astype(jnp.int32)

    ixmin = to_idx(xminf, w_img)
    ixmax = to_idx(xmaxf, w_img)
    iymin = to_idx(yminf, h_img)
    iymax = to_idx(ymaxf, h_img)
    iota = lax.broadcasted_iota(jnp.int32, (RT, E), 1)

    def oh(i):
        return (iota == i).astype(jnp.bfloat16)

    c = jnp.concatenate(
        [oh(ixmin) + oh(ixmax), oh(iymin) + oh(iymax),
         oh(ixmax - ixmin), oh(iymax - iymin)], axis=1)
    acc = jnp.dot(c, tp_ref[...], preferred_element_type=jnp.float32)
    o_ref[0] = jnp.maximum(acc, jnp.float32(0.0))


def _tc_half(bx, by, img_shapes, tp_bf):
    nb = bx.shape[0]
    nblk = bx.shape[1] // RT
    return pl.pallas_call(
        _tc_half_body,
        grid=(nb, nblk),
        in_specs=[
            pl.BlockSpec((1, RT, 4), lambda b, i: (b, i, 0)),
            pl.BlockSpec((1, RT, 4), lambda b, i: (b, i, 0)),
            pl.BlockSpec(memory_space=pltpu.SMEM),
            pl.BlockSpec((4 * E, D), lambda b, i: (0, 0)),
        ],
        out_specs=pl.BlockSpec((1, RT, D), lambda b, i: (b, i, 0)),
        out_shape=jax.ShapeDtypeStruct((nb, bx.shape[1], D), jnp.float32),
    )(bx, by, img_shapes, tp_bf)


def _sc_lookup_body(s0, n_sc_rows, n_per_batch, n_scales, coords_hbm, tp_hbm,
                    scales_hbm, out_hbm,
                    box0, box1, idx0, idx1, g0, g1, o0, o1, sc_v,
                    isem0, isem1, gsem0, gsem1, osem0, osem1):
    n_chunks = n_sc_rows // CH
    nb_floor = n_chunks // NW
    extra = n_chunks - nb_floor * NW
    wid = lax.axis_index("s") * NC + lax.axis_index("c")
    n_w = nb_floor + jnp.where(wid < extra, 1, 0)

    box = (box0, box1)
    idx = (idx0, idx1)
    gat = (g0, g1)
    out = (o0, o1)
    isem = (isem0, isem1)
    gsem = (gsem0, gsem1)
    osem = (osem0, osem1)

    pltpu.sync_copy(scales_hbm, sc_v.at[pl.ds(0, n_scales)])
    iota = lax.broadcasted_iota(jnp.int32, (L,), 0)
    ef = jnp.float32(E)
    emax = jnp.float32(E - 1)

    def row0(q):
        return (q * NW + wid) * CH

    def in_slice(r0):
        return coords_hbm.at[pl.ds(r0 * 8, CH * 8)]

    def in_start(r0, b):
        pltpu.async_copy(in_slice(r0), box[b], isem[b])

    def in_wait(r0, b):
        pltpu.make_async_copy(in_slice(r0), box[b], isem[b]).wait()

    def idx_compute(b, r0):
        for j in range(CH // L):
            batchg = (s0 + r0 + j * L) // n_per_batch
            h_img = plsc.load_gather(
                sc_v, [jnp.full((L,), 2 * batchg, jnp.int32)])
            w_img = plsc.load_gather(
                sc_v, [jnp.full((L,), 2 * batchg + 1, jnp.int32)])
            rows_k = (j * L + iota) * 8

            def coord(k):
                return plsc.load_gather(box[b], [rows_k + k])

            x0, x1, x2, x3 = coord(0), coord(2), coord(4), coord(6)
            y0, y1, y2, y3 = coord(1), coord(3), coord(5), coord(7)
            xminf = jnp.minimum(jnp.minimum(x0, x1), jnp.minimum(x2, x3))
            xmaxf = jnp.maximum(jnp.maximum(x0, x1), jnp.maximum(x2, x3))
            yminf = jnp.minimum(jnp.minimum(y0, y1), jnp.minimum(y2, y3))
            ymaxf = jnp.maximum(jnp.maximum(y0, y1), jnp.maximum(y2, y3))

            def to_idx(v, denom):
                scaled = (v / denom) * ef
                return jnp.clip(scaled, jnp.float32(0.0), emax).astype(jnp.int32)

            ixmin = to_idx(xminf, w_img)
            ixmax = to_idx(xmaxf, w_img)
            iymin = to_idx(yminf, h_img)
            iymax = to_idx(ymaxf, h_img)
            sl = pl.ds(j * L, L)
            idx[b][0, sl] = ixmin
            idx[b][1, sl] = iymin + E
            idx[b][2, sl] = ixmax
            idx[b][3, sl] = iymax + E
            idx[b][4, sl] = (ixmax - ixmin) + 2 * E
            idx[b][5, sl] = (iymax - iymin) + 3 * E

    def gather_start(b):
        for t in range(6):
            pltpu.async_copy(tp_hbm.at[idx[b].at[t]], gat[b].at[t], gsem[b])

    def gather_wait(b):
        for t in range(6):
            pltpu.make_async_copy(tp_hbm.at[idx[b].at[t]], gat[b].at[t],
                                  gsem[b]).wait()

    def accumulate(b):
        gv = gat[b]
        ov = out[b]

        @pl.loop(0, CH, unroll=2)
        def _acc(r):
            for w in range(D // L):
                s = pl.ds(w * L, L)
                acc = ((gv[0, r, s] + gv[1, r, s])
                       + (gv[2, r, s] + gv[3, r, s])
                       + (gv[4, r, s] + gv[5, r, s]))
                ov[r, s] = jnp.maximum(acc, jnp.float32(0.0))

    def out_start(r0, b):
        pltpu.async_copy(out[b], out_hbm.at[pl.ds(r0, CH)], osem[b])

    def out_wait(b):
        pltpu.make_async_copy(out[b], out_hbm.at[pl.ds(0, CH)],
                              osem[b]).wait()

    pltpu.sync_copy(in_slice(row0(0)), box[0])
    idx_compute(0, row0(0))
    gather_start(0)
    in_start(row0(1), 1)

    def step(q, p):
        @pl.when(q + 1 < n_w)
        def _():
            in_wait(row0(q + 1), 1 - p)
            idx_compute(1 - p, row0(q + 1))
            gather_start(1 - p)

            @pl.when(q + 2 < n_w)
            def _():
                in_start(row0(q + 2), p)

        gather_wait(p)

        @pl.when(q >= 2)
        def _():
            out_wait(p)

        accumulate(p)
        out_start(row0(q), p)

    @pl.loop(0, nb_floor // 2)
    def _pair(qq):
        step(2 * qq, 0)
        step(2 * qq + 1, 1)

    if nb_floor % 2:
        step(nb_floor - 1, (nb_floor - 1) % 2)

    @pl.when(wid < extra)
    def _():
        step(nb_floor, nb_floor % 2)

    out_wait(0)
    out_wait(1)


def kernel(boxes, img_shapes, x_emb, y_emb, w_emb, h_emb, W, b):
    B, N, K = boxes.shape
    n_rows = B * N
    tables = jnp.concatenate([x_emb, y_emb, w_emb, h_emb], axis=0)
    tp = _project_tables(tables, W, b)

    s0 = BS * N
    n_sc = n_rows - s0
    boxes_sc = boxes.reshape(n_rows, K)[s0:].reshape(n_sc * K)

    bx = boxes[:BS, :, 0::2]
    by = boxes[:BS, :, 1::2]
    out_tc = _tc_half(bx, by, img_shapes, tp.astype(jnp.bfloat16))

    mesh = plsc.VectorSubcoreMesh(core_axis_name="c", subcore_axis_name="s")
    body = functools.partial(_sc_lookup_body, s0, n_sc, N, B * 2)
    out_sc = pl.kernel(
        body,
        out_type=jax.ShapeDtypeStruct((n_sc, D), jnp.float32),
        mesh=mesh,
        compiler_params=pltpu.CompilerParams(needs_layout_passes=False),
        scratch_types=[
            pltpu.VMEM((CH * K,), jnp.float32),
            pltpu.VMEM((CH * K,), jnp.float32),
            pltpu.VMEM((6, CH), jnp.int32),
            pltpu.VMEM((6, CH), jnp.int32),
            pltpu.VMEM((6, CH, D), jnp.float32),
            pltpu.VMEM((6, CH, D), jnp.float32),
            pltpu.VMEM((CH, D), jnp.float32),
            pltpu.VMEM((CH, D), jnp.float32),
            pltpu.VMEM((max(B * 2, 128),), jnp.float32),
            pltpu.SemaphoreType.DMA,
            pltpu.SemaphoreType.DMA,
            pltpu.SemaphoreType.DMA,
            pltpu.SemaphoreType.DMA,
            pltpu.SemaphoreType.DMA,
            pltpu.SemaphoreType.DMA,
        ],
    )(boxes_sc, tp, img_shapes.reshape(B * 2))

    return jnp.concatenate(
        [out_tc, out_sc.reshape(B - BS, N, D)], axis=0)


# speedup vs baseline: 3.2056x; 1.0078x over previous
import functools

import jax
import jax.numpy as jnp
from jax import lax
from jax.experimental import pallas as pl
from jax.experimental.pallas import tpu as pltpu
from jax.experimental.pallas import tpu_sc as plsc

E = 128
D = 128
NC = 2
NS = 16
L = 16
NW = NC * NS
CH = 64
BS = 14
RT = 2000


def _project_tables_body(t_ref, w_ref, b_ref, out_ref):
    tp = lax.dot_general(
        t_ref[...], w_ref[...], (((1,), (1,)), ((), ())),
        preferred_element_type=jnp.float32)
    rows = lax.broadcasted_iota(jnp.int32, (4 * E, 1), 0)
    in_w_seg = (rows >= 2 * E) & (rows < 3 * E)
    out_ref[...] = tp + jnp.where(in_w_seg, b_ref[...], jnp.float32(0.0))


def _project_tables(tables, w, b):
    return pl.pallas_call(
        _project_tables_body,
        out_shape=jax.ShapeDtypeStruct((4 * E, D), jnp.float32),
    )(tables, w, b.reshape(1, D))


def _tc_half_body(xs_ref, ys_ref, scales_ref, tp_ref, o_ref):
    b = pl.program_id(0)
    sw = jnp.float32(E) / scales_ref[b, 1]
    sh = jnp.float32(E) / scales_ref[b, 0]
    xs = xs_ref[0]
    ys = ys_ref[0]
    xminf = jnp.min(xs, axis=1, keepdims=True)
    xmaxf = jnp.max(xs, axis=1, keepdims=True)
    yminf = jnp.min(ys, axis=1, keepdims=True)
    ymaxf = jnp.max(ys, axis=1, keepdims=True)

    def to_idx(v, scale):
        return jnp.clip(v * scale, jnp.float32(0.0),
                        jnp.float32(E - 1)).astype(jnp.int32)

    ixmin = to_idx(xminf, sw)
    ixmax = to_idx(xmaxf, sw)
    iymin = to_idx(yminf, sh)
    iymax = to_idx(ymaxf, sh)
    iota = lax.broadcasted_iota(jnp.int32, (RT, E), 1)

    def oh(i):
        return (iota == i).astype(jnp.bfloat16)

    c = jnp.concatenate(
        [oh(ixmin) + oh(ixmax), oh(iymin) + oh(iymax),
         oh(ixmax - ixmin), oh(iymax - iymin)], axis=1)
    acc = jnp.dot(c, tp_ref[...], preferred_element_type=jnp.float32)
    o_ref[0] = jnp.maximum(acc, jnp.float32(0.0))


def _tc_half(bx, by, img_shapes, tp_bf):
    nb = bx.shape[0]
    nblk = bx.shape[1] // RT
    return pl.pallas_call(
        _tc_half_body,
        grid=(nb, nblk),
        in_specs=[
            pl.BlockSpec((1, RT, 4), lambda b, i: (b, i, 0)),
            pl.BlockSpec((1, RT, 4), lambda b, i: (b, i, 0)),
            pl.BlockSpec(memory_space=pltpu.SMEM),
            pl.BlockSpec((4 * E, D), lambda b, i: (0, 0)),
        ],
        out_specs=pl.BlockSpec((1, RT, D), lambda b, i: (b, i, 0)),
        out_shape=jax.ShapeDtypeStruct((nb, bx.shape[1], D), jnp.float32),
    )(bx, by, img_shapes, tp_bf)


def _sc_lookup_body(s0, n_sc_rows, n_per_batch, n_scales, coords_hbm, tp_hbm,
                    scales_hbm, out_hbm,
                    box0, box1, idx0, idx1, g0, g1, o0, o1, sc_v,
                    isem0, isem1, gsem0, gsem1, osem0, osem1):
    n_chunks = n_sc_rows // CH
    nb_floor = n_chunks // NW
    extra = n_chunks - nb_floor * NW
    wid = lax.axis_index("s") * NC + lax.axis_index("c")
    n_w = nb_floor + jnp.where(wid < extra, 1, 0)

    box = (box0, box1)
    idx = (idx0, idx1)
    gat = (g0, g1)
    out = (o0, o1)
    isem = (isem0, isem1)
    gsem = (gsem0, gsem1)
    osem = (osem0, osem1)

    pltpu.sync_copy(scales_hbm, sc_v.at[pl.ds(0, n_scales)])
    iota = lax.broadcasted_iota(jnp.int32, (L,), 0)
    ef = jnp.float32(E)
    emax = jnp.float32(E - 1)

    def row0(q):
        return (q * NW + wid) * CH

    def in_slice(r0):
        return coords_hbm.at[pl.ds(r0 * 8, CH * 8)]

    def in_start(r0, b):
        pltpu.async_copy(in_slice(r0), box[b], isem[b])

    def in_wait(r0, b):
        pltpu.make_async_copy(in_slice(r0), box[b], isem[b]).wait()

    def idx_compute(b, r0):
        for j in range(CH // L):
            batchg = (s0 + r0 + j * L) // n_per_batch
            h_img = plsc.load_gather(
                sc_v, [jnp.full((L,), 2 * batchg, jnp.int32)])
            w_img = plsc.load_gather(
                sc_v, [jnp.full((L,), 2 * batchg + 1, jnp.int32)])
            rows_k = (j * L + iota) * 8

            def coord(k):
                return plsc.load_gather(box[b], [rows_k + k])

            x0, x1, x2, x3 = coord(0), coord(2), coord(4), coord(6)
            y0, y1, y2, y3 = coord(1), coord(3), coord(5), coord(7)
            xminf = jnp.minimum(jnp.minimum(x0, x1), jnp.minimum(x2, x3))
            xmaxf = jnp.maximum(jnp.maximum(x0, x1), jnp.maximum(x2, x3))
            yminf = jnp.minimum(jnp.minimum(y0, y1), jnp.minimum(y2, y3))
            ymaxf = jnp.maximum(jnp.maximum(y0, y1), jnp.maximum(y2, y3))

            def to_idx(v, denom):
                scaled = (v / denom) * ef
                return jnp.clip(scaled, jnp.float32(0.0), emax).astype(jnp.int32)

            ixmin = to_idx(xminf, w_img)
            ixmax = to_idx(xmaxf, w_img)
            iymin = to_idx(yminf, h_img)
            iymax = to_idx(ymaxf, h_img)
            sl = pl.ds(j * L, L)
            idx[b][0, sl] = ixmin
            idx[b][1, sl] = iymin + E
            idx[b][2, sl] = ixmax
            idx[b][3, sl] = iymax + E
            idx[b][4, sl] = (ixmax - ixmin) + 2 * E
            idx[b][5, sl] = (iymax - iymin) + 3 * E

    def gather_start(b):
        for t in range(6):
            pltpu.async_copy(tp_hbm.at[idx[b].at[t]], gat[b].at[t], gsem[b])

    def gather_wait(b):
        for t in range(6):
            pltpu.make_async_copy(tp_hbm.at[idx[b].at[t]], gat[b].at[t],
                                  gsem[b]).wait()

    def accumulate(b):
        gv = gat[b]
        ov = out[b]

        @pl.loop(0, CH, unroll=2)
        def _acc(r):
            for w in range(D // L):
                s = pl.ds(w * L, L)
                acc = ((gv[0, r, s] + gv[1, r, s])
                       + (gv[2, r, s] + gv[3, r, s])
                       + (gv[4, r, s] + gv[5, r, s]))
                ov[r, s] = jnp.maximum(acc, jnp.float32(0.0))

    def out_start(r0, b):
        pltpu.async_copy(out[b], out_hbm.at[pl.ds(r0, CH)], osem[b])

    def out_wait(b):
        pltpu.make_async_copy(out[b], out_hbm.at[pl.ds(0, CH)],
                              osem[b]).wait()

    pltpu.sync_copy(in_slice(row0(0)), box[0])
    idx_compute(0, row0(0))
    gather_start(0)
    in_start(row0(1), 1)

    def step(q, p):
        @pl.when(q + 1 < n_w)
        def _():
            in_wait(row0(q + 1), 1 - p)
            idx_compute(1 - p, row0(q + 1))
            gather_start(1 - p)

            @pl.when(q + 2 < n_w)
            def _():
                in_start(row0(q + 2), p)

        gather_wait(p)

        @pl.when(q >= 2)
        def _():
            out_wait(p)

        accumulate(p)
        out_start(row0(q), p)

    @pl.loop(0, nb_floor // 2)
    def _pair(qq):
        step(2 * qq, 0)
        step(2 * qq + 1, 1)

    if nb_floor % 2:
        step(nb_floor - 1, (nb_floor - 1) % 2)

    @pl.when(wid < extra)
    def _():
        step(nb_floor, nb_floor % 2)

    out_wait(0)
    out_wait(1)


def kernel(boxes, img_shapes, x_emb, y_emb, w_emb, h_emb, W, b):
    B, N, K = boxes.shape
    n_rows = B * N
    tables = jnp.concatenate([x_emb, y_emb, w_emb, h_emb], axis=0)
    tp = _project_tables(tables, W, b)

    s0 = BS * N
    n_sc = n_rows - s0
    boxes_sc = boxes.reshape(n_rows, K)[s0:].reshape(n_sc * K)

    bx = boxes[:BS, :, 0::2]
    by = boxes[:BS, :, 1::2]
    out_tc = _tc_half(bx, by, img_shapes, tp.astype(jnp.bfloat16))

    mesh = plsc.VectorSubcoreMesh(core_axis_name="c", subcore_axis_name="s")
    body = functools.partial(_sc_lookup_body, s0, n_sc, N, B * 2)
    out_sc = pl.kernel(
        body,
        out_type=jax.ShapeDtypeStruct((n_sc, D), jnp.float32),
        mesh=mesh,
        compiler_params=pltpu.CompilerParams(needs_layout_passes=False),
        scratch_types=[
            pltpu.VMEM((CH * K,), jnp.float32),
            pltpu.VMEM((CH * K,), jnp.float32),
            pltpu.VMEM((6, CH), jnp.int32),
            pltpu.VMEM((6, CH), jnp.int32),
            pltpu.VMEM((6, CH, D), jnp.float32),
            pltpu.VMEM((6, CH, D), jnp.float32),
            pltpu.VMEM((CH, D), jnp.float32),
            pltpu.VMEM((CH, D), jnp.float32),
            pltpu.VMEM((max(B * 2, 128),), jnp.float32),
            pltpu.SemaphoreType.DMA,
            pltpu.SemaphoreType.DMA,
            pltpu.SemaphoreType.DMA,
            pltpu.SemaphoreType.DMA,
            pltpu.SemaphoreType.DMA,
            pltpu.SemaphoreType.DMA,
        ],
    )(boxes_sc, tp, img_shapes.reshape(B * 2))

    return jnp.concatenate(
        [out_tc, out_sc.reshape(B - BS, N, D)], axis=0)
